# baseline (speedup 1.0000x reference)
import functools

import jax
import jax.numpy as jnp
from jax import lax
from jax.experimental import pallas as pl
from jax.experimental.pallas import tpu as pltpu
from jax.experimental.pallas import tpu_sc as plsc

_NC = 2
_NS = 16
_NW = _NC * _NS
_LANES = 16


def _pool_body(x_hbm, tab_hbm, h_hbm, idx_v, buf0, buf1, h_v, sem0, sem1,
               *, rpw, ctx, d, inv):
    wid = lax.axis_index("s") * _NC + lax.axis_index("c")
    base = wid * rpw
    pltpu.sync_copy(x_hbm.at[pl.ds(base, rpw)], idx_v)

    def start(r, buf, sem):
        pltpu.make_async_copy(tab_hbm.at[idx_v.at[r]], buf, sem).start()

    def wait(buf, sem):
        pltpu.make_async_copy(tab_hbm.at[idx_v.at[0]], buf, sem).wait()

    def reduce_row(buf, r):
        for v in range(d // _LANES):
            sl = pl.ds(v * _LANES, _LANES)
            acc = buf[0, sl]
            for j in range(1, ctx):
                acc = acc + buf[j, sl]
            h_v[r, sl] = acc * inv

    start(0, buf0, sem0)
    start(1, buf1, sem1)

    def body(i, carry):
        r = 2 * i
        wait(buf0, sem0)
        reduce_row(buf0, r)
        start(r + 2, buf0, sem0)
        wait(buf1, sem1)
        reduce_row(buf1, r + 1)
        start(r + 3, buf1, sem1)
        return carry

    lax.fori_loop(0, rpw // 2 - 1, body, 0)
    wait(buf0, sem0)
    reduce_row(buf0, rpw - 2)
    wait(buf1, sem1)
    reduce_row(buf1, rpw - 1)

    pltpu.sync_copy(h_v, h_hbm.at[pl.ds(base, rpw)])


def _pool(x, emb_table):
    b, ctx = x.shape
    _, d = emb_table.shape
    rpw = b // _NW
    mesh = plsc.VectorSubcoreMesh(core_axis_name="c", subcore_axis_name="s")
    body = functools.partial(_pool_body, rpw=rpw, ctx=ctx, d=d, inv=1.0 / ctx)
    return pl.kernel(
        body,
        out_type=jax.ShapeDtypeStruct((b, d), jnp.float32),
        mesh=mesh,
        scratch_types=[
            pltpu.VMEM((rpw, ctx), jnp.int32),
            pltpu.VMEM((ctx, d), jnp.float32),
            pltpu.VMEM((ctx, d), jnp.float32),
            pltpu.VMEM((rpw, d), jnp.float32),
            pltpu.SemaphoreType.DMA,
            pltpu.SemaphoreType.DMA,
        ],
    )(x, emb_table)


def _mm_body(h_ref, w_ref, o_ref):
    res = lax.dot_general(
        h_ref[...].astype(jnp.bfloat16), w_ref[...].astype(jnp.bfloat16),
        dimension_numbers=(((1,), (1,)), ((), ())),
        preferred_element_type=jnp.float32,
    )
    o_ref[...] = res.astype(jnp.bfloat16)


def _project(h, lin_w, bn=1024):
    b, d = h.shape
    v = lin_w.shape[0]
    grid = (pl.cdiv(v, bn),)
    return pl.pallas_call(
        _mm_body,
        grid=grid,
        in_specs=[
            pl.BlockSpec((b, d), lambda j: (0, 0)),
            pl.BlockSpec((bn, d), lambda j: (j, 0)),
        ],
        out_specs=pl.BlockSpec((b, bn), lambda j: (0, j)),
        out_shape=jax.ShapeDtypeStruct((b, v), jnp.bfloat16),
    )(h, lin_w)


def kernel(x, emb_table, lin_w):
    x = x.astype(jnp.int32)
    b = x.shape[0]
    h1 = _pool(x[: b // 2], emb_table)
    h2 = _pool(x[b // 2:], emb_table)
    l1 = _project(h1, lin_w)
    l2 = _project(h2, lin_w)
    return jnp.concatenate([l1, l2], axis=0).astype(jnp.float32)


# speedup vs baseline: 1.4434x; 1.4434x over previous
import functools

import jax
import jax.numpy as jnp
from jax import lax
from jax.experimental import pallas as pl
from jax.experimental.pallas import tpu as pltpu
from jax.experimental.pallas import tpu_sc as plsc

_NC = 2
_NS = 16
_NW = _NC * _NS
_LANES = 16


def _pool_body(x_hbm, tab_hbm, h_hbm, idx_v, buf0, buf1, h_v, sem0, sem1,
               *, rpw, ctx, d, inv):
    wid = lax.axis_index("s") * _NC + lax.axis_index("c")
    base = wid * rpw
    pltpu.sync_copy(x_hbm.at[pl.ds(base, rpw)], idx_v)

    def start(r, buf, sem):
        pltpu.make_async_copy(tab_hbm.at[idx_v.at[r]], buf, sem).start()

    def wait(buf, sem):
        pltpu.make_async_copy(tab_hbm.at[idx_v.at[0]], buf, sem).wait()

    def reduce_row(buf, r):
        for v in range(d // _LANES):
            sl = pl.ds(v * _LANES, _LANES)
            acc = buf[0, sl]
            for j in range(1, ctx):
                acc = acc + buf[j, sl]
            h_v[r, sl] = acc * inv

    start(0, buf0, sem0)
    start(1, buf1, sem1)

    def body(i, carry):
        r = 2 * i
        wait(buf0, sem0)
        reduce_row(buf0, r)
        start(r + 2, buf0, sem0)
        wait(buf1, sem1)
        reduce_row(buf1, r + 1)
        start(r + 3, buf1, sem1)
        return carry

    lax.fori_loop(0, rpw // 2 - 1, body, 0)
    wait(buf0, sem0)
    reduce_row(buf0, rpw - 2)
    wait(buf1, sem1)
    reduce_row(buf1, rpw - 1)

    pltpu.sync_copy(h_v, h_hbm.at[pl.ds(base, rpw)])


def _pool(x, emb_table):
    b, ctx = x.shape
    _, d = emb_table.shape
    rpw = b // _NW
    mesh = plsc.VectorSubcoreMesh(core_axis_name="c", subcore_axis_name="s")
    body = functools.partial(_pool_body, rpw=rpw, ctx=ctx, d=d, inv=1.0 / ctx)
    return pl.kernel(
        body,
        out_type=jax.ShapeDtypeStruct((b, d), jnp.float32),
        mesh=mesh,
        scratch_types=[
            pltpu.VMEM((rpw, ctx), jnp.int32),
            pltpu.VMEM((ctx, d), jnp.float32),
            pltpu.VMEM((ctx, d), jnp.float32),
            pltpu.VMEM((rpw, d), jnp.float32),
            pltpu.SemaphoreType.DMA,
            pltpu.SemaphoreType.DMA,
        ],
    )(x, emb_table)


def _mm_body(h_ref, w_ref, o_ref):
    res = lax.dot_general(
        h_ref[...].astype(jnp.bfloat16), w_ref[...].astype(jnp.bfloat16),
        dimension_numbers=(((1,), (1,)), ((), ())),
        preferred_element_type=jnp.float32,
    )
    o_ref[...] = res.astype(jnp.bfloat16)


def _project(h, lin_w, bn=1024):
    b, d = h.shape
    v = lin_w.shape[0]
    grid = (pl.cdiv(v, bn),)
    return pl.pallas_call(
        _mm_body,
        grid=grid,
        in_specs=[
            pl.BlockSpec((b, d), lambda j: (0, 0)),
            pl.BlockSpec((bn, d), lambda j: (j, 0)),
        ],
        out_specs=pl.BlockSpec((b, bn), lambda j: (0, j)),
        out_shape=jax.ShapeDtypeStruct((b, v), jnp.bfloat16),
    )(h, lin_w)


def kernel(x, emb_table, lin_w):
    x = x.astype(jnp.int32)
    h = _pool(x, emb_table)
    return _project(h, lin_w).astype(jnp.float32)


# speedup vs baseline: 1.4609x; 1.0121x over previous
import functools

import jax
import jax.numpy as jnp
from jax import lax
from jax.experimental import pallas as pl
from jax.experimental.pallas import tpu as pltpu
from jax.experimental.pallas import tpu_sc as plsc

_NC = 2
_NS = 16
_NW = _NC * _NS
_LANES = 16


def _pool_body(x_hbm, tab_hbm, h_hbm, idx_v, buf0, buf1, h_v, sem0, sem1,
               *, rpw, ctx, d, inv):
    wid = lax.axis_index("s") * _NC + lax.axis_index("c")
    base = wid * rpw
    pltpu.sync_copy(x_hbm.at[pl.ds(base, rpw)], idx_v)

    def start(r, buf, sem):
        pltpu.make_async_copy(tab_hbm.at[idx_v.at[r]], buf, sem).start()

    def wait(buf, sem):
        pltpu.make_async_copy(tab_hbm.at[idx_v.at[0]], buf, sem).wait()

    def reduce_row(buf, r):
        for v in range(d // _LANES):
            sl = pl.ds(v * _LANES, _LANES)
            acc = buf[0, sl]
            for j in range(1, ctx):
                acc = acc + buf[j, sl]
            h_v[r, sl] = acc * inv

    start(0, buf0, sem0)
    start(1, buf1, sem1)

    def body(i, carry):
        r = 2 * i
        wait(buf0, sem0)
        reduce_row(buf0, r)
        start(r + 2, buf0, sem0)
        wait(buf1, sem1)
        reduce_row(buf1, r + 1)
        start(r + 3, buf1, sem1)
        return carry

    lax.fori_loop(0, rpw // 2 - 1, body, 0)
    wait(buf0, sem0)
    reduce_row(buf0, rpw - 2)
    wait(buf1, sem1)
    reduce_row(buf1, rpw - 1)

    pltpu.sync_copy(h_v, h_hbm.at[pl.ds(base, rpw)])


def _pool(x, emb_table):
    b, ctx = x.shape
    _, d = emb_table.shape
    rpw = b // _NW
    mesh = plsc.VectorSubcoreMesh(core_axis_name="c", subcore_axis_name="s")
    body = functools.partial(_pool_body, rpw=rpw, ctx=ctx, d=d, inv=1.0 / ctx)
    return pl.kernel(
        body,
        out_type=jax.ShapeDtypeStruct((b, d), jnp.float32),
        mesh=mesh,
        scratch_types=[
            pltpu.VMEM((rpw, ctx), jnp.int32),
            pltpu.VMEM((ctx, d), jnp.float32),
            pltpu.VMEM((ctx, d), jnp.float32),
            pltpu.VMEM((rpw, d), jnp.float32),
            pltpu.SemaphoreType.DMA,
            pltpu.SemaphoreType.DMA,
        ],
    )(x, emb_table)


def _mm_body(h_ref, w_ref, o_ref):
    res = lax.dot_general(
        h_ref[...].astype(jnp.bfloat16), w_ref[...].astype(jnp.bfloat16),
        dimension_numbers=(((1,), (1,)), ((), ())),
        preferred_element_type=jnp.float32,
    )
    o_ref[...] = res.astype(jnp.bfloat16)


def _project(h, lin_w, bn=2048):
    b, d = h.shape
    v = lin_w.shape[0]
    grid = (pl.cdiv(v, bn),)
    return pl.pallas_call(
        _mm_body,
        grid=grid,
        in_specs=[
            pl.BlockSpec((b, d), lambda j: (0, 0)),
            pl.BlockSpec((bn, d), lambda j: (j, 0)),
        ],
        out_specs=pl.BlockSpec((b, bn), lambda j: (0, j)),
        out_shape=jax.ShapeDtypeStruct((b, v), jnp.bfloat16),
    )(h, lin_w)


def kernel(x, emb_table, lin_w):
    x = x.astype(jnp.int32)
    h = _pool(x, emb_table)
    return _project(h, lin_w).astype(jnp.float32)


# speedup vs baseline: 1.4611x; 1.0002x over previous
import functools

import jax
import jax.numpy as jnp
from jax import lax
from jax.experimental import pallas as pl
from jax.experimental.pallas import tpu as pltpu
from jax.experimental.pallas import tpu_sc as plsc

_NC = 2
_NS = 16
_NW = _NC * _NS
_LANES = 16


def _pool_body(x_hbm, tab_hbm, h_hbm, idx_v, buf0, buf1, h_v, sem0, sem1,
               *, rpw, ctx, d, inv):
    wid = lax.axis_index("s") * _NC + lax.axis_index("c")
    base = wid * rpw
    pltpu.sync_copy(x_hbm.at[pl.ds(base, rpw)], idx_v)

    def start(r, buf, sem):
        pltpu.make_async_copy(tab_hbm.at[idx_v.at[r]], buf, sem).start()

    def wait(buf, sem):
        pltpu.make_async_copy(tab_hbm.at[idx_v.at[0]], buf, sem).wait()

    def reduce_row(buf, r):
        for v in range(d // _LANES):
            sl = pl.ds(v * _LANES, _LANES)
            acc = buf[0, sl]
            for j in range(1, ctx):
                acc = acc + buf[j, sl]
            h_v[r, sl] = acc * inv

    start(0, buf0, sem0)
    start(1, buf1, sem1)

    def body(i, carry):
        r = 2 * i
        wait(buf0, sem0)
        reduce_row(buf0, r)
        start(r + 2, buf0, sem0)
        wait(buf1, sem1)
        reduce_row(buf1, r + 1)
        start(r + 3, buf1, sem1)
        return carry

    lax.fori_loop(0, rpw // 2 - 1, body, 0)
    wait(buf0, sem0)
    reduce_row(buf0, rpw - 2)
    wait(buf1, sem1)
    reduce_row(buf1, rpw - 1)

    pltpu.sync_copy(h_v, h_hbm.at[pl.ds(base, rpw)])


def _pool(x, emb_table):
    b, ctx = x.shape
    _, d = emb_table.shape
    rpw = b // _NW
    mesh = plsc.VectorSubcoreMesh(core_axis_name="c", subcore_axis_name="s")
    body = functools.partial(_pool_body, rpw=rpw, ctx=ctx, d=d, inv=1.0 / ctx)
    return pl.kernel(
        body,
        out_type=jax.ShapeDtypeStruct((b, d), jnp.float32),
        mesh=mesh,
        scratch_types=[
            pltpu.VMEM((rpw, ctx), jnp.int32),
            pltpu.VMEM((ctx, d), jnp.float32),
            pltpu.VMEM((ctx, d), jnp.float32),
            pltpu.VMEM((rpw, d), jnp.float32),
            pltpu.SemaphoreType.DMA,
            pltpu.SemaphoreType.DMA,
        ],
    )(x, emb_table)


def _mm_body(h_ref, w_ref, o_ref):
    res = lax.dot_general(
        h_ref[...].astype(jnp.bfloat16), w_ref[...].astype(jnp.bfloat16),
        dimension_numbers=(((1,), (1,)), ((), ())),
        preferred_element_type=jnp.float32,
    )
    o_ref[...] = res.astype(jnp.bfloat16)


def _project(h, lin_w, bn=3072):
    b, d = h.shape
    v = lin_w.shape[0]
    grid = (pl.cdiv(v, bn),)
    return pl.pallas_call(
        _mm_body,
        grid=grid,
        in_specs=[
            pl.BlockSpec((b, d), lambda j: (0, 0)),
            pl.BlockSpec((bn, d), lambda j: (j, 0)),
        ],
        out_specs=pl.BlockSpec((b, bn), lambda j: (0, j)),
        out_shape=jax.ShapeDtypeStruct((b, v), jnp.bfloat16),
    )(h, lin_w)


def kernel(x, emb_table, lin_w):
    x = x.astype(jnp.int32)
    h = _pool(x, emb_table)
    return _project(h, lin_w).astype(jnp.float32)
